# baseline (speedup 1.0000x reference)
import functools

import jax
import jax.numpy as jnp
from jax import lax
from jax.experimental import pallas as pl
from jax.experimental.pallas import tpu as pltpu
from jax.experimental.pallas import tpu_sc as plsc

RESO = 128
STEP = 0.01
N_STEPS = 256
N_RAYS = 16384

NW = 32
CHUNK_R = 16
NVEC = N_STEPS // 16
GRID_WORDS = RESO * RESO * RESO // 32


def _sc_sample(code, grid_words, ts_tab, te_tab, rid0, n_rows):
    rows_per_w = n_rows // NW
    n_chunks = rows_per_w // CHUNK_R
    mesh = plsc.VectorSubcoreMesh(core_axis_name="c", subcore_axis_name="s")

    @functools.partial(
        pl.kernel,
        mesh=mesh,
        compiler_params=pltpu.CompilerParams(needs_layout_passes=False),
        out_type=(
            jax.ShapeDtypeStruct((n_rows, N_STEPS), jnp.int32),
            jax.ShapeDtypeStruct((n_rows, N_STEPS), jnp.float32),
            jax.ShapeDtypeStruct((n_rows, N_STEPS), jnp.float32),
        ),
        scratch_types=[
            pltpu.VMEM((GRID_WORDS,), jnp.int32),
            pltpu.VMEM((N_STEPS,), jnp.float32),
            pltpu.VMEM((N_STEPS,), jnp.float32),
            pltpu.VMEM((2, CHUNK_R, N_STEPS), jnp.int32),
            pltpu.VMEM((2, CHUNK_R, N_STEPS), jnp.int32),
            pltpu.VMEM((2, CHUNK_R, N_STEPS), jnp.float32),
            pltpu.VMEM((2, CHUNK_R, N_STEPS), jnp.float32),
            pltpu.SemaphoreType.DMA,
            pltpu.SemaphoreType.DMA,
            pltpu.SemaphoreType.DMA,
            pltpu.SemaphoreType.DMA,
        ],
    )
    def k(code_hbm, grid_hbm, tst_hbm, tet_hbm, ri_hbm, ts_hbm, te_hbm,
          grid_v, tst_v, tet_v, cbuf, ribuf, tsbuf, tebuf,
          insem0, insem1, outsem0, outsem1):
        wid = lax.axis_index("s") * 2 + lax.axis_index("c")
        base0 = wid * rows_per_w
        insems = (insem0, insem1)
        outsems = (outsem0, outsem1)

        def in_copy(cc, b):
            return pltpu.make_async_copy(
                code_hbm.at[pl.ds(base0 + cc * CHUNK_R, CHUNK_R)],
                cbuf.at[b], insems[b])

        def out_copies(cc, b):
            sl = pl.ds(base0 + cc * CHUNK_R, CHUNK_R)
            return (pltpu.make_async_copy(ribuf.at[b], ri_hbm.at[sl], outsems[b]),
                    pltpu.make_async_copy(tsbuf.at[b], ts_hbm.at[sl], outsems[b]),
                    pltpu.make_async_copy(tebuf.at[b], te_hbm.at[sl], outsems[b]))

        in_copy(0, 0).start()
        pltpu.sync_copy(grid_hbm, grid_v)
        pltpu.sync_copy(tst_hbm, tst_v)
        pltpu.sync_copy(tet_hbm, tet_v)
        tsvs = [tst_v[pl.ds(v * 16, 16)] for v in range(NVEC)]
        tevs = [tet_v[pl.ds(v * 16, 16)] for v in range(NVEC)]

        def compute_chunk(cc, b):
            rowbase = base0 + cc * CHUNK_R

            @plsc.parallel_loop(0, CHUNK_R, 1, unroll=2)
            def row_body(r):
                ridv = jnp.full((16,), rid0 + rowbase + r, dtype=jnp.int32)
                for v in range(NVEC):
                    sl = pl.ds(v * 16, 16)
                    cd = cbuf[b, r, sl]
                    word = plsc.load_gather(grid_v, [cd >> 6])
                    m = ((word >> ((cd >> 1) & 31)) & cd & 1) == 1
                    ribuf[b, r, sl] = jnp.where(m, ridv, -1)
                    tsbuf[b, r, sl] = jnp.where(m, tsvs[v], 0.0)
                    tebuf[b, r, sl] = jnp.where(m, tevs[v], 0.0)

        def step(i, b):
            cc = i * 2 + b

            @pl.when(cc < n_chunks - 1)
            def _():
                in_copy(cc + 1, b ^ 1).start()

            in_copy(cc, b).wait()

            @pl.when(i >= 1)
            def _():
                for h in out_copies(cc - 2, b):
                    h.wait()

            compute_chunk(cc, b)
            for h in out_copies(cc, b):
                h.start()

        def body2(i, carry):
            step(i, 0)
            step(i, 1)
            return carry

        lax.fori_loop(0, n_chunks // 2, body2, 0)
        for h in out_copies(n_chunks - 2, 0):
            h.wait()
        for h in out_copies(n_chunks - 1, 1):
            h.wait()

    return k(code, grid_words, ts_tab, te_tab)


def _code_half(rays_o, d, t_mid, aabb):
    pos = rays_o[:, None, :] + d[:, None, :] * t_mid[None, :, None]
    size = aabb[1] - aabb[0]
    g = (pos - aabb[0][None, None, :]) / size[None, None, :] * RESO
    idx = jnp.clip(g.astype(jnp.int32), 0, RESO - 1)
    inside = jnp.all((pos >= aabb[0][None, None, :])
                     & (pos < aabb[1][None, None, :]), axis=-1)
    widx = idx[..., 0] * 512 + idx[..., 1] * 4 + (idx[..., 2] >> 5)
    return (widx << 6) | ((idx[..., 2] & 31) << 1) | inside.astype(jnp.int32)


def kernel(rays_o, rays_d, occ_grid, aabb, near_far):
    d = rays_d / (jnp.linalg.norm(rays_d, axis=-1, keepdims=True) + 1e-8)
    t_mid = near_far[0] + (jnp.arange(N_STEPS, dtype=jnp.float32) + 0.5) * STEP
    gw = occ_grid.reshape(-1, 32).astype(jnp.uint32)
    words = (gw << jnp.arange(32, dtype=jnp.uint32)[None, :]).sum(
        axis=1, dtype=jnp.uint32)
    words = lax.bitcast_convert_type(words, jnp.int32)
    tst = t_mid - 0.5 * STEP
    tet = t_mid + 0.5 * STEP
    half = N_RAYS // 2
    outs = []
    for h in range(2):
        sl = slice(h * half, (h + 1) * half)
        code_h = _code_half(rays_o[sl], d[sl], t_mid, aabb)
        outs.append(_sc_sample(code_h, words, tst, tet, h * half, half))
    ri = jnp.concatenate([outs[0][0], outs[1][0]], axis=0)
    ts = jnp.concatenate([outs[0][1], outs[1][1]], axis=0)
    te = jnp.concatenate([outs[0][2], outs[1][2]], axis=0)
    return ri, ts, te, ri >= 0


# speedup vs baseline: 1.2099x; 1.2099x over previous
import functools

import jax
import jax.numpy as jnp
from jax import lax
from jax.experimental import pallas as pl
from jax.experimental.pallas import tpu as pltpu
from jax.experimental.pallas import tpu_sc as plsc

RESO = 128
STEP = 0.01
N_STEPS = 256
N_RAYS = 16384

NW = 32
CHUNK_R = 16
NVEC = N_STEPS // 16
GRID_WORDS = RESO * RESO * RESO // 32


def _sc_sample(code, grid_words, ts_tab, te_tab, rid0, n_rows):
    rows_per_w = n_rows // NW
    n_chunks = rows_per_w // CHUNK_R
    mesh = plsc.VectorSubcoreMesh(core_axis_name="c", subcore_axis_name="s")

    @functools.partial(
        pl.kernel,
        mesh=mesh,
        compiler_params=pltpu.CompilerParams(needs_layout_passes=False),
        out_type=(
            jax.ShapeDtypeStruct((n_rows, N_STEPS), jnp.int32),
            jax.ShapeDtypeStruct((n_rows, N_STEPS), jnp.float32),
            jax.ShapeDtypeStruct((n_rows, N_STEPS), jnp.float32),
        ),
        scratch_types=[
            pltpu.VMEM((GRID_WORDS,), jnp.int32),
            pltpu.VMEM((N_STEPS,), jnp.float32),
            pltpu.VMEM((N_STEPS,), jnp.float32),
            pltpu.VMEM((2, CHUNK_R, N_STEPS), jnp.int32),
            pltpu.VMEM((2, CHUNK_R, N_STEPS), jnp.int32),
            pltpu.VMEM((2, CHUNK_R, N_STEPS), jnp.float32),
            pltpu.VMEM((2, CHUNK_R, N_STEPS), jnp.float32),
            pltpu.SemaphoreType.DMA,
            pltpu.SemaphoreType.DMA,
            pltpu.SemaphoreType.DMA,
            pltpu.SemaphoreType.DMA,
        ],
    )
    def k(code_hbm, grid_hbm, tst_hbm, tet_hbm, ri_hbm, ts_hbm, te_hbm,
          grid_v, tst_v, tet_v, cbuf, ribuf, tsbuf, tebuf,
          insem0, insem1, outsem0, outsem1):
        wid = lax.axis_index("s") * 2 + lax.axis_index("c")
        base0 = wid * rows_per_w
        insems = (insem0, insem1)
        outsems = (outsem0, outsem1)

        def in_copy(cc, b):
            return pltpu.make_async_copy(
                code_hbm.at[pl.ds(base0 + cc * CHUNK_R, CHUNK_R)],
                cbuf.at[b], insems[b])

        def out_copies(cc, b):
            sl = pl.ds(base0 + cc * CHUNK_R, CHUNK_R)
            return (pltpu.make_async_copy(ribuf.at[b], ri_hbm.at[sl], outsems[b]),
                    pltpu.make_async_copy(tsbuf.at[b], ts_hbm.at[sl], outsems[b]),
                    pltpu.make_async_copy(tebuf.at[b], te_hbm.at[sl], outsems[b]))

        in_copy(0, 0).start()
        pltpu.sync_copy(grid_hbm, grid_v)
        pltpu.sync_copy(tst_hbm, tst_v)
        pltpu.sync_copy(tet_hbm, tet_v)
        tsvs = [tst_v[pl.ds(v * 16, 16)] for v in range(NVEC)]
        tevs = [tet_v[pl.ds(v * 16, 16)] for v in range(NVEC)]

        def compute_chunk(cc, b):
            rowbase = base0 + cc * CHUNK_R

            @plsc.parallel_loop(0, CHUNK_R, 1, unroll=2)
            def row_body(r):
                ridv = jnp.full((16,), rid0 + rowbase + r, dtype=jnp.int32)
                for v in range(NVEC):
                    sl = pl.ds(v * 16, 16)
                    cd = cbuf[b, r, sl]
                    word = plsc.load_gather(grid_v, [cd >> 6])
                    m = ((word >> ((cd >> 1) & 31)) & cd & 1) == 1
                    ribuf[b, r, sl] = jnp.where(m, ridv, -1)
                    tsbuf[b, r, sl] = jnp.where(m, tsvs[v], 0.0)
                    tebuf[b, r, sl] = jnp.where(m, tevs[v], 0.0)

        def step(i, b):
            cc = i * 2 + b

            @pl.when(cc < n_chunks - 1)
            def _():
                in_copy(cc + 1, b ^ 1).start()

            in_copy(cc, b).wait()

            @pl.when(i >= 1)
            def _():
                for h in out_copies(cc - 2, b):
                    h.wait()

            compute_chunk(cc, b)
            for h in out_copies(cc, b):
                h.start()

        def body2(i, carry):
            step(i, 0)
            step(i, 1)
            return carry

        lax.fori_loop(0, n_chunks // 2, body2, 0)
        for h in out_copies(n_chunks - 2, 0):
            h.wait()
        for h in out_copies(n_chunks - 1, 1):
            h.wait()

    return k(code, grid_words, ts_tab, te_tab)


def _code_half(rays_o, d, t_mid, aabb):
    pos = rays_o[:, None, :] + d[:, None, :] * t_mid[None, :, None]
    size = aabb[1] - aabb[0]
    g = (pos - aabb[0][None, None, :]) / size[None, None, :] * RESO
    idx = jnp.clip(g.astype(jnp.int32), 0, RESO - 1)
    inside = jnp.all((pos >= aabb[0][None, None, :])
                     & (pos < aabb[1][None, None, :]), axis=-1)
    widx = idx[..., 0] * 512 + idx[..., 1] * 4 + (idx[..., 2] >> 5)
    return (widx << 6) | ((idx[..., 2] & 31) << 1) | inside.astype(jnp.int32)


def kernel(rays_o, rays_d, occ_grid, aabb, near_far):
    d = rays_d / (jnp.linalg.norm(rays_d, axis=-1, keepdims=True) + 1e-8)
    t_mid = near_far[0] + (jnp.arange(N_STEPS, dtype=jnp.float32) + 0.5) * STEP
    gw = occ_grid.reshape(-1, 32).astype(jnp.uint32)
    words = (gw << jnp.arange(32, dtype=jnp.uint32)[None, :]).sum(
        axis=1, dtype=jnp.uint32)
    words = lax.bitcast_convert_type(words, jnp.int32)
    tst = t_mid - 0.5 * STEP
    tet = t_mid + 0.5 * STEP
    code = _code_half(rays_o, d, t_mid, aabb)
    ri, ts, te = _sc_sample(code, words, tst, tet, 0, N_RAYS)
    return ri, ts, te, ri >= 0


# speedup vs baseline: 1.3142x; 1.0862x over previous
import functools

import jax
import jax.numpy as jnp
from jax import lax
from jax.experimental import pallas as pl
from jax.experimental.pallas import tpu as pltpu
from jax.experimental.pallas import tpu_sc as plsc

RESO = 128
STEP = 0.01
N_STEPS = 256
N_RAYS = 16384

NW = 32
CHUNK_R = 16
NVEC = N_STEPS // 16
GRID_WORDS = RESO * RESO * RESO // 32


def _sc_sample(code, grid_words, ts_tab, te_tab, rid0, n_rows):
    rows_per_w = n_rows // NW
    n_chunks = rows_per_w // CHUNK_R
    mesh = plsc.VectorSubcoreMesh(core_axis_name="c", subcore_axis_name="s")

    @functools.partial(
        pl.kernel,
        mesh=mesh,
        compiler_params=pltpu.CompilerParams(needs_layout_passes=False),
        out_type=(
            jax.ShapeDtypeStruct((n_rows, N_STEPS), jnp.int32),
            jax.ShapeDtypeStruct((n_rows, N_STEPS), jnp.float32),
            jax.ShapeDtypeStruct((n_rows, N_STEPS), jnp.float32),
        ),
        scratch_types=[
            pltpu.VMEM((GRID_WORDS,), jnp.int32),
            pltpu.VMEM((N_STEPS,), jnp.float32),
            pltpu.VMEM((N_STEPS,), jnp.float32),
            pltpu.VMEM((2, CHUNK_R, N_STEPS), jnp.int32),
            pltpu.VMEM((2, CHUNK_R, N_STEPS), jnp.int32),
            pltpu.VMEM((2, CHUNK_R, N_STEPS), jnp.float32),
            pltpu.VMEM((2, CHUNK_R, N_STEPS), jnp.float32),
            pltpu.SemaphoreType.DMA,
            pltpu.SemaphoreType.DMA,
            pltpu.SemaphoreType.DMA,
            pltpu.SemaphoreType.DMA,
        ],
    )
    def k(code_hbm, grid_hbm, tst_hbm, tet_hbm, ri_hbm, ts_hbm, te_hbm,
          grid_v, tst_v, tet_v, cbuf, ribuf, tsbuf, tebuf,
          insem0, insem1, outsem0, outsem1):
        wid = lax.axis_index("s") * 2 + lax.axis_index("c")
        base0 = wid * rows_per_w
        insems = (insem0, insem1)
        outsems = (outsem0, outsem1)

        def in_copy(cc, b):
            return pltpu.make_async_copy(
                code_hbm.at[pl.ds(base0 + cc * CHUNK_R, CHUNK_R)],
                cbuf.at[b], insems[b])

        def out_copies(cc, b):
            sl = pl.ds(base0 + cc * CHUNK_R, CHUNK_R)
            return (pltpu.make_async_copy(ribuf.at[b], ri_hbm.at[sl], outsems[b]),
                    pltpu.make_async_copy(tsbuf.at[b], ts_hbm.at[sl], outsems[b]),
                    pltpu.make_async_copy(tebuf.at[b], te_hbm.at[sl], outsems[b]))

        in_copy(0, 0).start()
        pltpu.sync_copy(grid_hbm, grid_v)
        pltpu.sync_copy(tst_hbm, tst_v)
        pltpu.sync_copy(tet_hbm, tet_v)
        tsvs = [tst_v[pl.ds(v * 16, 16)] for v in range(NVEC)]
        tevs = [tet_v[pl.ds(v * 16, 16)] for v in range(NVEC)]

        def compute_chunk(cc, b):
            rowbase = base0 + cc * CHUNK_R

            @plsc.parallel_loop(0, CHUNK_R, 1, unroll=4)
            def row_body(r):
                ridv = jnp.full((16,), rid0 + rowbase + r, dtype=jnp.int32)
                for v in range(NVEC):
                    sl = pl.ds(v * 16, 16)
                    cd = cbuf[b, r, sl]
                    word = plsc.load_gather(grid_v, [cd >> 6])
                    m = ((word >> ((cd >> 1) & 31)) & cd & 1) == 1
                    ribuf[b, r, sl] = jnp.where(m, ridv, -1)
                    tsbuf[b, r, sl] = jnp.where(m, tsvs[v], 0.0)
                    tebuf[b, r, sl] = jnp.where(m, tevs[v], 0.0)

        def step(i, b):
            cc = i * 2 + b

            @pl.when(cc < n_chunks - 1)
            def _():
                in_copy(cc + 1, b ^ 1).start()

            in_copy(cc, b).wait()

            @pl.when(i >= 1)
            def _():
                for h in out_copies(cc - 2, b):
                    h.wait()

            compute_chunk(cc, b)
            for h in out_copies(cc, b):
                h.start()

        def body2(i, carry):
            step(i, 0)
            step(i, 1)
            return carry

        lax.fori_loop(0, n_chunks // 2, body2, 0)
        for h in out_copies(n_chunks - 2, 0):
            h.wait()
        for h in out_copies(n_chunks - 1, 1):
            h.wait()

    return k(code, grid_words, ts_tab, te_tab)


def _code_half(rays_o, d, t_mid, aabb):
    pos = rays_o[:, None, :] + d[:, None, :] * t_mid[None, :, None]
    size = aabb[1] - aabb[0]
    g = (pos - aabb[0][None, None, :]) / size[None, None, :] * RESO
    idx = jnp.clip(g.astype(jnp.int32), 0, RESO - 1)
    inside = jnp.all((pos >= aabb[0][None, None, :])
                     & (pos < aabb[1][None, None, :]), axis=-1)
    widx = idx[..., 0] * 512 + idx[..., 1] * 4 + (idx[..., 2] >> 5)
    return (widx << 6) | ((idx[..., 2] & 31) << 1) | inside.astype(jnp.int32)


def kernel(rays_o, rays_d, occ_grid, aabb, near_far):
    d = rays_d / (jnp.linalg.norm(rays_d, axis=-1, keepdims=True) + 1e-8)
    t_mid = near_far[0] + (jnp.arange(N_STEPS, dtype=jnp.float32) + 0.5) * STEP
    gw = occ_grid.reshape(-1, 32).astype(jnp.uint32)
    words = (gw << jnp.arange(32, dtype=jnp.uint32)[None, :]).sum(
        axis=1, dtype=jnp.uint32)
    words = lax.bitcast_convert_type(words, jnp.int32)
    tst = t_mid - 0.5 * STEP
    tet = t_mid + 0.5 * STEP
    code = _code_half(rays_o, d, t_mid, aabb)
    ri, ts, te = _sc_sample(code, words, tst, tet, 0, N_RAYS)
    return ri, ts, te, ri >= 0


# speedup vs baseline: 1.4272x; 1.0860x over previous
import functools

import jax
import jax.numpy as jnp
from jax import lax
from jax.experimental import pallas as pl
from jax.experimental.pallas import tpu as pltpu
from jax.experimental.pallas import tpu_sc as plsc

RESO = 128
STEP = 0.01
N_STEPS = 256
N_RAYS = 16384

NW = 32
CHUNK_R = 16
NVEC = N_STEPS // 16
GRID_WORDS = RESO * RESO * RESO // 32


def _sc_sample(code, grid_words, ts_tab, te_tab, rid0, n_rows):
    rows_per_w = n_rows // NW
    n_chunks = rows_per_w // CHUNK_R
    mesh = plsc.VectorSubcoreMesh(core_axis_name="c", subcore_axis_name="s")

    @functools.partial(
        pl.kernel,
        mesh=mesh,
        compiler_params=pltpu.CompilerParams(needs_layout_passes=False),
        out_type=(
            jax.ShapeDtypeStruct((n_rows, N_STEPS), jnp.int32),
            jax.ShapeDtypeStruct((n_rows, N_STEPS), jnp.float32),
            jax.ShapeDtypeStruct((n_rows, N_STEPS), jnp.float32),
        ),
        scratch_types=[
            pltpu.VMEM((GRID_WORDS,), jnp.int32),
            pltpu.VMEM((N_STEPS,), jnp.float32),
            pltpu.VMEM((N_STEPS,), jnp.float32),
            pltpu.VMEM((2, CHUNK_R, N_STEPS), jnp.int32),
            pltpu.VMEM((2, CHUNK_R, N_STEPS), jnp.int32),
            pltpu.VMEM((2, CHUNK_R, N_STEPS), jnp.float32),
            pltpu.VMEM((2, CHUNK_R, N_STEPS), jnp.float32),
            pltpu.SemaphoreType.DMA,
            pltpu.SemaphoreType.DMA,
            pltpu.SemaphoreType.DMA,
            pltpu.SemaphoreType.DMA,
        ],
    )
    def k(code_hbm, grid_hbm, tst_hbm, tet_hbm, ri_hbm, ts_hbm, te_hbm,
          grid_v, tst_v, tet_v, cbuf, ribuf, tsbuf, tebuf,
          insem0, insem1, outsem0, outsem1):
        wid = lax.axis_index("s") * 2 + lax.axis_index("c")
        base0 = wid * rows_per_w
        insems = (insem0, insem1)
        outsems = (outsem0, outsem1)

        def in_copy(cc, b):
            return pltpu.make_async_copy(
                code_hbm.at[pl.ds(base0 + cc * CHUNK_R, CHUNK_R)],
                cbuf.at[b], insems[b])

        def out_copies(cc, b):
            sl = pl.ds(base0 + cc * CHUNK_R, CHUNK_R)
            return (pltpu.make_async_copy(ribuf.at[b], ri_hbm.at[sl], outsems[b]),
                    pltpu.make_async_copy(tsbuf.at[b], ts_hbm.at[sl], outsems[b]),
                    pltpu.make_async_copy(tebuf.at[b], te_hbm.at[sl], outsems[b]))

        in_copy(0, 0).start()
        pltpu.sync_copy(grid_hbm, grid_v)
        pltpu.sync_copy(tst_hbm, tst_v)
        pltpu.sync_copy(tet_hbm, tet_v)
        tsvs = [tst_v[pl.ds(v * 16, 16)] for v in range(NVEC)]
        tevs = [tet_v[pl.ds(v * 16, 16)] for v in range(NVEC)]

        def compute_chunk(cc, b):
            rowbase = base0 + cc * CHUNK_R

            @plsc.parallel_loop(0, CHUNK_R, 1, unroll=8)
            def row_body(r):
                ridv = jnp.full((16,), rid0 + rowbase + r, dtype=jnp.int32)
                for v in range(NVEC):
                    sl = pl.ds(v * 16, 16)
                    cd = cbuf[b, r, sl]
                    word = plsc.load_gather(grid_v, [cd >> 6])
                    m = ((word >> ((cd >> 1) & 31)) & cd & 1) == 1
                    ribuf[b, r, sl] = jnp.where(m, ridv, -1)
                    tsbuf[b, r, sl] = jnp.where(m, tsvs[v], 0.0)
                    tebuf[b, r, sl] = jnp.where(m, tevs[v], 0.0)

        def step(i, b):
            cc = i * 2 + b

            @pl.when(cc < n_chunks - 1)
            def _():
                in_copy(cc + 1, b ^ 1).start()

            in_copy(cc, b).wait()

            @pl.when(i >= 1)
            def _():
                for h in out_copies(cc - 2, b):
                    h.wait()

            compute_chunk(cc, b)
            for h in out_copies(cc, b):
                h.start()

        def body2(i, carry):
            step(i, 0)
            step(i, 1)
            return carry

        lax.fori_loop(0, n_chunks // 2, body2, 0)
        for h in out_copies(n_chunks - 2, 0):
            h.wait()
        for h in out_copies(n_chunks - 1, 1):
            h.wait()

    return k(code, grid_words, ts_tab, te_tab)


def _code_half(rays_o, d, t_mid, aabb):
    pos = rays_o[:, None, :] + d[:, None, :] * t_mid[None, :, None]
    size = aabb[1] - aabb[0]
    g = (pos - aabb[0][None, None, :]) / size[None, None, :] * RESO
    idx = jnp.clip(g.astype(jnp.int32), 0, RESO - 1)
    inside = jnp.all((pos >= aabb[0][None, None, :])
                     & (pos < aabb[1][None, None, :]), axis=-1)
    widx = idx[..., 0] * 512 + idx[..., 1] * 4 + (idx[..., 2] >> 5)
    return (widx << 6) | ((idx[..., 2] & 31) << 1) | inside.astype(jnp.int32)


def kernel(rays_o, rays_d, occ_grid, aabb, near_far):
    d = rays_d / (jnp.linalg.norm(rays_d, axis=-1, keepdims=True) + 1e-8)
    t_mid = near_far[0] + (jnp.arange(N_STEPS, dtype=jnp.float32) + 0.5) * STEP
    gw = occ_grid.reshape(-1, 32).astype(jnp.uint32)
    words = (gw << jnp.arange(32, dtype=jnp.uint32)[None, :]).sum(
        axis=1, dtype=jnp.uint32)
    words = lax.bitcast_convert_type(words, jnp.int32)
    tst = t_mid - 0.5 * STEP
    tet = t_mid + 0.5 * STEP
    code = _code_half(rays_o, d, t_mid, aabb)
    ri, ts, te = _sc_sample(code, words, tst, tet, 0, N_RAYS)
    return ri, ts, te, ri >= 0
